# Initial kernel scaffold; baseline (speedup 1.0000x reference)
import jax
import jax.numpy as jnp
from jax.experimental import pallas as pl


def kernel(snoRNAs, diseases, emb_sno, emb_dis, W_fc, b_fc, W_fcg, b_fcg, graph_rows, graph_cols, graph_vals):
    raise NotImplementedError("write your pallas kernel here")


# speedup vs baseline: 4.1876x; 4.1876x over previous
import functools

import jax
import jax.numpy as jnp
from jax import lax
from jax.experimental import pallas as pl
from jax.experimental.pallas import tpu as pltpu
from jax.experimental.pallas import tpu_sc as plsc

N_SNO = 50000
N_DIS = 10000
T = N_SNO + N_DIS
D = 200
G = 4
E = 400000
B = 4096

DP = 208
NV = DP // 16
R = 1024
NCH = 60
TP = NCH * R
DRB = 4096
K = 128
WRK = 32
EPT = 12928
EPP = WRK * EPT
STR = R // 16
ACC_ROWS = R + 16

_MESH = dict(core_axis_name="c", subcore_axis_name="s")


def _zero_rows(buf, nrows):
  zv = jnp.zeros((16,), jnp.float32)

  def body(r, _):
    for v in range(NV):
      buf[r, pl.ds(v * 16, 16)] = zv
    return _

  lax.fori_loop(0, nrows, body, None)


def _make_spmm(ng, with_acc):
  n_in = 2 * ng + 4 + (1 if with_acc else 0)
  n_out = ng + (1 if with_acc else 0)
  out_type = [jax.ShapeDtypeStruct((TP, DP), jnp.float32)] * n_out
  scratch = [
      pltpu.SMEM((64,), jnp.int32),
      pltpu.SMEM((64,), jnp.int32),
      pltpu.VMEM((K,), jnp.int32),
      pltpu.VMEM((K,), jnp.int32),
      pltpu.VMEM((K,), jnp.int32),
      pltpu.VMEM((K,), jnp.float32),
      pltpu.VMEM((K, DP), jnp.float32),
      pltpu.VMEM((STR, DP), jnp.float32),
      pltpu.VMEM((STR, DP), jnp.float32),
  ] + [pltpu.VMEM_SHARED((ACC_ROWS, DP), jnp.float32) for _ in range(ng)]

  def body(*refs):
    ins = refs[:n_in]
    outs = refs[n_in:n_in + n_out]
    (sA, sE, rbuf, cbuf, lbuf, wbuf, stage, wb, wbsum) = refs[n_in + n_out:
                                                             n_in + n_out + 9]
    accs = refs[n_in + n_out + 9:]
    cur_in = ins[:ng]
    wgs = ins[ng:2 * ng]
    rows_h, cols_h, offA_h, offE_h = ins[2 * ng:2 * ng + 4]
    cur_out = outs[:ng]

    cid = lax.axis_index("c")
    sid = lax.axis_index("s")
    pltpu.sync_copy(offA_h, rbuf.at[pl.ds(0, 64)])
    pltpu.sync_copy(offE_h, cbuf.at[pl.ds(0, 64)])
    for jb in range(4):
      av = rbuf[pl.ds(jb * 16, 16)]
      ev = cbuf[pl.ds(jb * 16, 16)]
      for jj in range(16):
        sA[jb * 16 + jj] = av[jj]
        sE[jb * 16 + jj] = ev[jj]
    _zero_rows(wbsum, STR)

    def chunk_body(i, _):
      c = 2 * i + cid
      oa = sA[c]
      oe = sE[c + 1]
      n = oe - oa
      pt = ((n + 15) // 16 + (K - 1)) // K * K
      nb = pt // K
      rowbase = c * R

      for g in range(ng):
        pltpu.sync_copy(wbsum.at[pl.ds(0, STR)],
                        accs[g].at[pl.ds(sid * STR, STR)])
      plsc.subcore_barrier()

      def batch_body(b, _):
        ebase = pl.multiple_of(oa + sid * pt + b * K, 8)
        pltpu.sync_copy(rows_h.at[pl.ds(ebase, K)], rbuf)
        pltpu.sync_copy(cols_h.at[pl.ds(ebase, K)], cbuf)

        def lidx_body(jb, _):
          rv = rbuf[pl.ds(jb * 16, 16)] - rowbase
          ok = (rv >= 0) & (rv < R)
          lbuf[pl.ds(jb * 16, 16)] = jnp.where(ok, rv, R)
          return _

        lax.fori_loop(0, K // 16, lidx_body, None)

        for g in range(ng):
          pltpu.sync_copy(wgs[g].at[pl.ds(ebase, K)], wbuf)
          pltpu.sync_copy(cur_in[g].at[cbuf], stage)

          def scale_body(jb, _):
            wv = wbuf[pl.ds(jb * 16, 16)]
            for jj in range(16):
              w = wv[jj]
              j = jb * 16 + jj
              for v in range(NV):
                stage[j, pl.ds(v * 16, 16)] = stage[j, pl.ds(v * 16, 16)] * w
            return _

          lax.fori_loop(0, K // 16, scale_body, None)
          pltpu.sync_copy(stage, accs[g].at[lbuf], add=True)
        return _

      lax.fori_loop(0, nb, batch_body, None)
      plsc.subcore_barrier()

      r0 = sid * STR
      gr0 = pl.multiple_of(rowbase + r0, 8)
      for g in range(ng):
        pltpu.sync_copy(accs[g].at[pl.ds(r0, STR)], wb)
        pltpu.sync_copy(wb, cur_out[g].at[pl.ds(gr0, STR)])
        if with_acc:

          def sum_body(r, _):
            for v in range(NV):
              sl = pl.ds(v * 16, 16)
              if g == 0:
                wbsum[r, sl] = wb[r, sl]
              else:
                wbsum[r, sl] = wbsum[r, sl] + wb[r, sl]
            return _

          lax.fori_loop(0, STR, sum_body, None)
      if with_acc:
        oacc_in = ins[n_in - 1]
        oacc_out = outs[n_out - 1]
        pltpu.sync_copy(oacc_in.at[pl.ds(gr0, STR)], wb)

        def oacc_body(r, _):
          for v in range(NV):
            sl = pl.ds(v * 16, 16)
            wb[r, sl] = wb[r, sl] + wbsum[r, sl]
            wbsum[r, sl] = jnp.zeros((16,), jnp.float32)
          return _

        lax.fori_loop(0, STR, oacc_body, None)
        pltpu.sync_copy(wb, oacc_out.at[pl.ds(gr0, STR)])
      plsc.subcore_barrier()
      return _

    lax.fori_loop(0, NCH // 2, chunk_body, None)

  return pl.kernel(body, out_type=out_type,
                   mesh=plsc.VectorSubcoreMesh(**_MESH),
                   compiler_params=pltpu.CompilerParams(
                       use_tc_tiling_on_sc=False, needs_layout_passes=False),
                   scratch_types=scratch)


def _make_wkernel():
  out_type = [jax.ShapeDtypeStruct((EPP,), jnp.float32)] * G
  scratch = (
      [pltpu.VMEM((K,), jnp.int32) for _ in range(4)]
      + [pltpu.VMEM((K,), jnp.float32)]
      + [pltpu.VMEM((K,), jnp.float32) for _ in range(G)]
  )

  def body(gid_h, rows_h, cols_h, vals_h, w0, w1, w2, w3, rbuf, cbuf, grb,
           gcb, vbuf, o0, o1, o2, o3):
    wouts = (w0, w1, w2, w3)
    obufs = (o0, o1, o2, o3)
    cid = lax.axis_index("c")
    sid = lax.axis_index("s")
    wid = sid * 2 + cid
    base = wid * EPT

    def batch_body(b, _):
      ebase = pl.multiple_of(base + b * K, 8)
      pltpu.sync_copy(rows_h.at[pl.ds(ebase, K)], rbuf)
      pltpu.sync_copy(cols_h.at[pl.ds(ebase, K)], cbuf)
      pltpu.sync_copy(vals_h.at[pl.ds(ebase, K)], vbuf)
      pltpu.sync_copy(gid_h.at[rbuf], grb)
      pltpu.sync_copy(gid_h.at[cbuf], gcb)

      def grp_body(jb, _):
        sl = pl.ds(jb * 16, 16)
        gr = grb[sl]
        gc = gcb[sl]
        vv = vbuf[sl]
        for g in range(G):
          m = ((gr >> g) & 1) * ((gc >> g) & 1)
          obufs[g][sl] = vv * m.astype(jnp.float32)
        return _

      lax.fori_loop(0, K // 16, grp_body, None)
      for g in range(G):
        pltpu.sync_copy(obufs[g], wouts[g].at[pl.ds(ebase, K)])
      return _

    lax.fori_loop(0, EPT // K, batch_body, None)

  return pl.kernel(body, out_type=out_type,
                   mesh=plsc.VectorSubcoreMesh(**_MESH),
                   compiler_params=pltpu.CompilerParams(
                       use_tc_tiling_on_sc=False, needs_layout_passes=False),
                   scratch_types=scratch)


def _make_final():
  out_type = [jax.ShapeDtypeStruct((B,), jnp.float32)]
  scratch = [
      pltpu.VMEM((K,), jnp.int32),
      pltpu.VMEM((K,), jnp.int32),
      pltpu.VMEM((K, DP), jnp.float32),
      pltpu.VMEM((K, DP), jnp.float32),
      pltpu.VMEM((K,), jnp.float32),
  ]

  def body(acc_h, sno_h, dis_h, gamma_h, sibuf, dibuf, sstage, dstage, gbuf):
    lanes = lax.iota(jnp.int32, 16)
    cid = lax.axis_index("c")
    sid = lax.axis_index("s")
    wid = sid * 2 + cid
    base = pl.multiple_of(wid * K, 8)
    pltpu.sync_copy(sno_h.at[pl.ds(base, K)], sibuf)
    pltpu.sync_copy(dis_h.at[pl.ds(base, K)], dibuf)

    def shift_body(jb, _):
      sl = pl.ds(jb * 16, 16)
      dibuf[sl] = dibuf[sl] + N_SNO
      return _

    lax.fori_loop(0, K // 16, shift_body, None)
    pltpu.sync_copy(acc_h.at[sibuf], sstage)
    pltpu.sync_copy(acc_h.at[dibuf], dstage)

    def dot_body(jb, _):
      gvec = jnp.zeros((16,), jnp.float32)
      for jj in range(16):
        j = jb * 16 + jj
        acc = sstage[j, pl.ds(0, 16)] * dstage[j, pl.ds(0, 16)]
        for v in range(1, NV):
          sl = pl.ds(v * 16, 16)
          acc = acc + sstage[j, sl] * dstage[j, sl]
        tot = jnp.sum(acc) * 0.04
        gvec = jnp.where(lanes == jj, tot, gvec)
      gbuf[pl.ds(jb * 16, 16)] = gvec
      return _

    lax.fori_loop(0, K // 16, dot_body, None)
    pltpu.sync_copy(gbuf, gamma_h.at[pl.ds(base, K)])

  return pl.kernel(body, out_type=out_type,
                   mesh=plsc.VectorSubcoreMesh(**_MESH),
                   compiler_params=pltpu.CompilerParams(
                       use_tc_tiling_on_sc=False, needs_layout_passes=False),
                   scratch_types=scratch)


def _dense_body(ego_ref, side_ref, wfc_ref, bfc_ref, wfcg_ref, bfcg_ref,
                gid_ref, acc0_ref):
  i = pl.program_id(0)
  x = ego_ref[:, :D] + side_ref[:, :D]
  t = jnp.dot(x, wfc_ref[...], preferred_element_type=jnp.float32)
  t = t + bfc_ref[...]
  t = jnp.where(t >= 0, t, t * jnp.float32(0.01))
  s = jnp.dot(t, wfcg_ref[...], preferred_element_type=jnp.float32)
  s = s + bfcg_ref[...]
  mx = jnp.max(s, axis=1, keepdims=True)
  hot = (s == mx).astype(jnp.int32)
  bits = (hot[:, 0:1] + 2 * hot[:, 1:2] + 4 * hot[:, 2:3] + 8 * hot[:, 3:4])
  row = i * DRB + lax.broadcasted_iota(jnp.int32, (DRB, 1), 0)
  gid_ref[...] = jnp.where(row >= N_SNO, 15, bits)
  acc0_ref[...] = ego_ref[...] * jnp.float32(G)


_dense = pl.pallas_call(
    _dense_body,
    grid=(TP // DRB,),
    in_specs=[
        pl.BlockSpec((DRB, DP), lambda i: (i, 0)),
        pl.BlockSpec((DRB, DP), lambda i: (i, 0)),
        pl.BlockSpec((D, D), lambda i: (0, 0)),
        pl.BlockSpec((1, D), lambda i: (0, 0)),
        pl.BlockSpec((D, G), lambda i: (0, 0)),
        pl.BlockSpec((1, G), lambda i: (0, 0)),
    ],
    out_specs=[
        pl.BlockSpec((DRB, 1), lambda i: (i, 0)),
        pl.BlockSpec((DRB, DP), lambda i: (i, 0)),
    ],
    out_shape=[
        jax.ShapeDtypeStruct((TP, 1), jnp.int32),
        jax.ShapeDtypeStruct((TP, DP), jnp.float32),
    ],
)


def kernel(snoRNAs, diseases, emb_sno, emb_dis, W_fc, b_fc, W_fcg, b_fcg,
           graph_rows, graph_cols, graph_vals):
  all_emb = jnp.concatenate([emb_sno, emb_dis], axis=0)
  ego = jnp.pad(all_emb, ((0, TP - T), (0, DP - D)))
  order = jnp.argsort(graph_rows)
  rows_s = jnp.pad(graph_rows[order].astype(jnp.int32), (0, EPP - E),
                   constant_values=TP - 1)
  cols_s = jnp.pad(graph_cols[order].astype(jnp.int32), (0, EPP - E))
  vals_s = jnp.pad(graph_vals[order], (0, EPP - E))
  bounds = jnp.arange(NCH + 1, dtype=jnp.int32) * R
  off = jnp.searchsorted(rows_s, bounds, side="left").astype(jnp.int32)
  offA = jnp.pad((off // 8) * 8, (0, 63 - NCH))
  offE = jnp.pad(jnp.minimum(off, E), (0, 63 - NCH))

  spmm1 = _make_spmm(1, False)
  (side,) = spmm1(ego, vals_s, rows_s, cols_s, offA, offE)

  gid2d, outacc = _dense(ego, side, W_fc, b_fc.reshape(1, D), W_fcg,
                         b_fcg.reshape(1, G))
  gid = gid2d.reshape(TP)

  wk = _make_wkernel()
  w0, w1, w2, w3 = wk(gid, rows_s, cols_s, vals_s)

  rk = _make_spmm(G, True)
  cur = (ego, ego, ego, ego)
  for _ in range(4):
    res = rk(cur[0], cur[1], cur[2], cur[3], w0, w1, w2, w3, rows_s, cols_s,
             offA, offE, outacc)
    cur = res[:G]
    outacc = res[G]

  fk = _make_final()
  (gamma,) = fk(outacc, snoRNAs.astype(jnp.int32), diseases.astype(jnp.int32))
  return gamma


# speedup vs baseline: 5.3094x; 1.2679x over previous
import functools

import jax
import jax.numpy as jnp
from jax import lax
from jax.experimental import pallas as pl
from jax.experimental.pallas import tpu as pltpu
from jax.experimental.pallas import tpu_sc as plsc

N_SNO = 50000
N_DIS = 10000
T = N_SNO + N_DIS
D = 200
G = 4
E = 400000
B = 4096

DP = 208
NV = DP // 16
R = 768
NCH = 80
TP = NCH * R
DRB = 4096
K = 128
WRK = 32
EPT = 12928
EPP = WRK * EPT
STR = R // 16
ACC_ROWS = R + 16

_MESH = dict(core_axis_name="c", subcore_axis_name="s")


def _zero_rows(buf, nrows):
  zv = jnp.zeros((16,), jnp.float32)

  def body(r, _):
    for v in range(NV):
      buf[r, pl.ds(v * 16, 16)] = zv
    return _

  lax.fori_loop(0, nrows, body, None)


def _make_spmm(ng, with_acc):
  n_in = 2 * ng + 4 + (1 if with_acc else 0)
  n_out = ng + (1 if with_acc else 0)
  out_type = [jax.ShapeDtypeStruct((TP, DP), jnp.float32)] * n_out
  scratch = ([
      pltpu.SMEM((96,), jnp.int32),
      pltpu.SMEM((96,), jnp.int32),
      pltpu.VMEM((K,), jnp.int32),
      pltpu.VMEM((K,), jnp.int32),
      pltpu.VMEM((K,), jnp.int32),
  ] + [pltpu.VMEM((K,), jnp.float32) for _ in range(ng)]
    + [pltpu.VMEM((K, DP), jnp.float32) for _ in range(2)]
    + [
      pltpu.VMEM((STR, DP), jnp.float32),
      pltpu.VMEM((STR, DP), jnp.float32),
  ] + [pltpu.VMEM_SHARED((ACC_ROWS, DP), jnp.float32) for _ in range(ng)]
    + [pltpu.SemaphoreType.DMA for _ in range(4)])

  def body(*refs):
    ins = refs[:n_in]
    outs = refs[n_in:n_in + n_out]
    scr = refs[n_in + n_out:]
    sA, sE, rbuf, cbuf, lbuf = scr[:5]
    wbufs = scr[5:5 + ng]
    stages = scr[5 + ng:7 + ng]
    wb, wbsum = scr[7 + ng], scr[8 + ng]
    accs = scr[9 + ng:9 + 2 * ng]
    gsems = scr[9 + 2 * ng:11 + 2 * ng]
    ssems = scr[11 + 2 * ng:13 + 2 * ng]
    cur_in = ins[:ng]
    wgs = ins[ng:2 * ng]
    rows_h, cols_h, offA_h, offE_h = ins[2 * ng:2 * ng + 4]
    cur_out = outs[:ng]

    cid = lax.axis_index("c")
    sid = lax.axis_index("s")
    pltpu.sync_copy(offA_h, rbuf.at[pl.ds(0, 96)])
    pltpu.sync_copy(offE_h, cbuf.at[pl.ds(0, 96)])
    for jb in range(6):
      av = rbuf[pl.ds(jb * 16, 16)]
      ev = cbuf[pl.ds(jb * 16, 16)]
      for jj in range(16):
        sA[jb * 16 + jj] = av[jj]
        sE[jb * 16 + jj] = ev[jj]
    _zero_rows(wbsum, STR)

    def chunk_body(i, _):
      c = 2 * i + cid
      oa = sA[c]
      oe = sE[c + 1]
      n = oe - oa
      pt = ((n + 15) // 16 + (K - 1)) // K * K
      nb = pt // K
      rowbase = c * R

      for g in range(ng):
        pltpu.sync_copy(wbsum.at[pl.ds(0, STR)],
                        accs[g].at[pl.ds(sid * STR, STR)])
      plsc.subcore_barrier()

      def drain_scatter(g, u):
        pltpu.make_async_copy(stages[u], accs[g].at[lbuf], ssems[u]).wait()

      def batch_body(b, _):
        @pl.when(b > 0)
        def _():
          drain_scatter(ng - 2 if ng > 1 else 0, 0)
          if ng > 1:
            drain_scatter(ng - 1, 1)

        ebase = pl.multiple_of(oa + sid * pt + b * K, 8)
        pltpu.sync_copy(cols_h.at[pl.ds(ebase, K)], cbuf)
        pltpu.async_copy(cur_in[0].at[cbuf], stages[0], gsems[0])
        pltpu.sync_copy(rows_h.at[pl.ds(ebase, K)], rbuf)

        def lidx_body(jb, _):
          rv = rbuf[pl.ds(jb * 16, 16)] - rowbase
          ok = (rv >= 0) & (rv < R)
          lbuf[pl.ds(jb * 16, 16)] = jnp.where(ok, rv, R)
          return _

        lax.fori_loop(0, K // 16, lidx_body, None)
        for g in range(ng):
          pltpu.sync_copy(wgs[g].at[pl.ds(ebase, K)], wbufs[g])

        for g in range(ng):
          u = g % 2
          if g + 1 < ng:
            un = (g + 1) % 2
            if g + 1 >= 2:
              drain_scatter(g - 1, un)
            pltpu.async_copy(cur_in[g + 1].at[cbuf], stages[un], gsems[un])
          pltpu.make_async_copy(cur_in[g].at[cbuf], stages[u],
                                gsems[u]).wait()

          def scale_body(jb, _, g=g, u=u):
            wv = wbufs[g][pl.ds(jb * 16, 16)]
            for jj in range(16):
              w = wv[jj]
              j = jb * 16 + jj
              for v in range(NV):
                stages[u][j, pl.ds(v * 16, 16)] = (
                    stages[u][j, pl.ds(v * 16, 16)] * w)
            return _

          lax.fori_loop(0, K // 16, scale_body, None)
          pltpu.async_copy(stages[u], accs[g].at[lbuf], ssems[u], add=True)
        return _

      lax.fori_loop(0, nb, batch_body, None)

      @pl.when(nb > 0)
      def _():
        drain_scatter(ng - 2 if ng > 1 else 0, 0)
        if ng > 1:
          drain_scatter(ng - 1, 1)

      plsc.subcore_barrier()

      r0 = sid * STR
      gr0 = pl.multiple_of(rowbase + r0, 8)
      for g in range(ng):
        pltpu.sync_copy(accs[g].at[pl.ds(r0, STR)], wb)
        pltpu.sync_copy(wb, cur_out[g].at[pl.ds(gr0, STR)])
        if with_acc:

          def sum_body(r, _):
            for v in range(NV):
              sl = pl.ds(v * 16, 16)
              if g == 0:
                wbsum[r, sl] = wb[r, sl]
              else:
                wbsum[r, sl] = wbsum[r, sl] + wb[r, sl]
            return _

          lax.fori_loop(0, STR, sum_body, None)
      if with_acc:
        oacc_in = ins[n_in - 1]
        oacc_out = outs[n_out - 1]
        pltpu.sync_copy(oacc_in.at[pl.ds(gr0, STR)], wb)

        def oacc_body(r, _):
          for v in range(NV):
            sl = pl.ds(v * 16, 16)
            wb[r, sl] = wb[r, sl] + wbsum[r, sl]
            wbsum[r, sl] = jnp.zeros((16,), jnp.float32)
          return _

        lax.fori_loop(0, STR, oacc_body, None)
        pltpu.sync_copy(wb, oacc_out.at[pl.ds(gr0, STR)])
      plsc.subcore_barrier()
      return _

    lax.fori_loop(0, NCH // 2, chunk_body, None)

  return pl.kernel(body, out_type=out_type,
                   mesh=plsc.VectorSubcoreMesh(**_MESH),
                   compiler_params=pltpu.CompilerParams(
                       use_tc_tiling_on_sc=False, needs_layout_passes=False),
                   scratch_types=scratch)


def _make_wkernel():
  out_type = [jax.ShapeDtypeStruct((EPP,), jnp.float32)] * G
  scratch = (
      [pltpu.VMEM((K,), jnp.int32) for _ in range(4)]
      + [pltpu.VMEM((K,), jnp.float32)]
      + [pltpu.VMEM((K,), jnp.float32) for _ in range(G)]
  )

  def body(gid_h, rows_h, cols_h, vals_h, w0, w1, w2, w3, rbuf, cbuf, grb,
           gcb, vbuf, o0, o1, o2, o3):
    wouts = (w0, w1, w2, w3)
    obufs = (o0, o1, o2, o3)
    cid = lax.axis_index("c")
    sid = lax.axis_index("s")
    wid = sid * 2 + cid
    base = wid * EPT

    def batch_body(b, _):
      ebase = pl.multiple_of(base + b * K, 8)
      pltpu.sync_copy(rows_h.at[pl.ds(ebase, K)], rbuf)
      pltpu.sync_copy(cols_h.at[pl.ds(ebase, K)], cbuf)
      pltpu.sync_copy(vals_h.at[pl.ds(ebase, K)], vbuf)
      pltpu.sync_copy(gid_h.at[rbuf], grb)
      pltpu.sync_copy(gid_h.at[cbuf], gcb)

      def grp_body(jb, _):
        sl = pl.ds(jb * 16, 16)
        gr = grb[sl]
        gc = gcb[sl]
        vv = vbuf[sl]
        for g in range(G):
          m = ((gr >> g) & 1) * ((gc >> g) & 1)
          obufs[g][sl] = vv * m.astype(jnp.float32)
        return _

      lax.fori_loop(0, K // 16, grp_body, None)
      for g in range(G):
        pltpu.sync_copy(obufs[g], wouts[g].at[pl.ds(ebase, K)])
      return _

    lax.fori_loop(0, EPT // K, batch_body, None)

  return pl.kernel(body, out_type=out_type,
                   mesh=plsc.VectorSubcoreMesh(**_MESH),
                   compiler_params=pltpu.CompilerParams(
                       use_tc_tiling_on_sc=False, needs_layout_passes=False),
                   scratch_types=scratch)


def _make_final():
  out_type = [jax.ShapeDtypeStruct((B,), jnp.float32)]
  scratch = [
      pltpu.VMEM((K,), jnp.int32),
      pltpu.VMEM((K,), jnp.int32),
      pltpu.VMEM((K, DP), jnp.float32),
      pltpu.VMEM((K, DP), jnp.float32),
      pltpu.VMEM((K,), jnp.float32),
  ]

  def body(acc_h, sno_h, dis_h, gamma_h, sibuf, dibuf, sstage, dstage, gbuf):
    lanes = lax.iota(jnp.int32, 16)
    cid = lax.axis_index("c")
    sid = lax.axis_index("s")
    wid = sid * 2 + cid
    base = pl.multiple_of(wid * K, 8)
    pltpu.sync_copy(sno_h.at[pl.ds(base, K)], sibuf)
    pltpu.sync_copy(dis_h.at[pl.ds(base, K)], dibuf)

    def shift_body(jb, _):
      sl = pl.ds(jb * 16, 16)
      dibuf[sl] = dibuf[sl] + N_SNO
      return _

    lax.fori_loop(0, K // 16, shift_body, None)
    pltpu.sync_copy(acc_h.at[sibuf], sstage)
    pltpu.sync_copy(acc_h.at[dibuf], dstage)

    def dot_body(jb, _):
      gvec = jnp.zeros((16,), jnp.float32)
      for jj in range(16):
        j = jb * 16 + jj
        acc = sstage[j, pl.ds(0, 16)] * dstage[j, pl.ds(0, 16)]
        for v in range(1, NV):
          sl = pl.ds(v * 16, 16)
          acc = acc + sstage[j, sl] * dstage[j, sl]
        tot = jnp.sum(acc) * 0.04
        gvec = jnp.where(lanes == jj, tot, gvec)
      gbuf[pl.ds(jb * 16, 16)] = gvec
      return _

    lax.fori_loop(0, K // 16, dot_body, None)
    pltpu.sync_copy(gbuf, gamma_h.at[pl.ds(base, K)])

  return pl.kernel(body, out_type=out_type,
                   mesh=plsc.VectorSubcoreMesh(**_MESH),
                   compiler_params=pltpu.CompilerParams(
                       use_tc_tiling_on_sc=False, needs_layout_passes=False),
                   scratch_types=scratch)


def _dense_body(ego_ref, side_ref, wfc_ref, bfc_ref, wfcg_ref, bfcg_ref,
                gid_ref, acc0_ref):
  i = pl.program_id(0)
  x = ego_ref[:, :D] + side_ref[:, :D]
  t = jnp.dot(x, wfc_ref[...], preferred_element_type=jnp.float32)
  t = t + bfc_ref[...]
  t = jnp.where(t >= 0, t, t * jnp.float32(0.01))
  s = jnp.dot(t, wfcg_ref[...], preferred_element_type=jnp.float32)
  s = s + bfcg_ref[...]
  mx = jnp.max(s, axis=1, keepdims=True)
  hot = (s == mx).astype(jnp.int32)
  bits = (hot[:, 0:1] + 2 * hot[:, 1:2] + 4 * hot[:, 2:3] + 8 * hot[:, 3:4])
  row = i * DRB + lax.broadcasted_iota(jnp.int32, (DRB, 1), 0)
  gid_ref[...] = jnp.where(row >= N_SNO, 15, bits)
  acc0_ref[...] = ego_ref[...] * jnp.float32(G)


_dense = pl.pallas_call(
    _dense_body,
    grid=(TP // DRB,),
    in_specs=[
        pl.BlockSpec((DRB, DP), lambda i: (i, 0)),
        pl.BlockSpec((DRB, DP), lambda i: (i, 0)),
        pl.BlockSpec((D, D), lambda i: (0, 0)),
        pl.BlockSpec((1, D), lambda i: (0, 0)),
        pl.BlockSpec((D, G), lambda i: (0, 0)),
        pl.BlockSpec((1, G), lambda i: (0, 0)),
    ],
    out_specs=[
        pl.BlockSpec((DRB, 1), lambda i: (i, 0)),
        pl.BlockSpec((DRB, DP), lambda i: (i, 0)),
    ],
    out_shape=[
        jax.ShapeDtypeStruct((TP, 1), jnp.int32),
        jax.ShapeDtypeStruct((TP, DP), jnp.float32),
    ],
)


def kernel(snoRNAs, diseases, emb_sno, emb_dis, W_fc, b_fc, W_fcg, b_fcg,
           graph_rows, graph_cols, graph_vals):
  all_emb = jnp.concatenate([emb_sno, emb_dis], axis=0)
  ego = jnp.pad(all_emb, ((0, TP - T), (0, DP - D)))
  order = jnp.argsort(graph_rows)
  rows_s = jnp.pad(graph_rows[order].astype(jnp.int32), (0, EPP - E),
                   constant_values=TP - 1)
  cols_s = jnp.pad(graph_cols[order].astype(jnp.int32), (0, EPP - E))
  vals_s = jnp.pad(graph_vals[order], (0, EPP - E))
  bounds = jnp.arange(NCH + 1, dtype=jnp.int32) * R
  off = jnp.searchsorted(rows_s, bounds, side="left").astype(jnp.int32)
  offA = jnp.pad((off // 8) * 8, (0, 95 - NCH))
  offE = jnp.pad(jnp.minimum(off, E), (0, 95 - NCH))

  spmm1 = _make_spmm(1, False)
  (side,) = spmm1(ego, vals_s, rows_s, cols_s, offA, offE)

  gid2d, outacc = _dense(ego, side, W_fc, b_fc.reshape(1, D), W_fcg,
                         b_fcg.reshape(1, G))
  gid = gid2d.reshape(TP)

  wk = _make_wkernel()
  w0, w1, w2, w3 = wk(gid, rows_s, cols_s, vals_s)

  rk = _make_spmm(G, True)
  cur = (ego, ego, ego, ego)
  for _ in range(4):
    res = rk(cur[0], cur[1], cur[2], cur[3], w0, w1, w2, w3, rows_s, cols_s,
             offA, offE, outacc)
    cur = res[:G]
    outacc = res[G]

  fk = _make_final()
  (gamma,) = fk(outacc, snoRNAs.astype(jnp.int32), diseases.astype(jnp.int32))
  return gamma
